# Initial kernel scaffold; baseline (speedup 1.0000x reference)
import jax
import jax.numpy as jnp
from jax.experimental import pallas as pl


def kernel(xg_old, x, batch, Wg, bg, Wf, bf, Wt, bt):
    raise NotImplementedError("write your pallas kernel here")


# speedup vs baseline: 10.5563x; 10.5563x over previous
import jax
import jax.numpy as jnp
from jax import lax
from jax.experimental import pallas as pl
from jax.experimental.pallas import tpu as pltpu


def _leaky(v):
    return jnp.where(v >= 0, v, 0.01 * v)


def _fused_body(x_ref, batch_ref, wg_ref, bg_ref, wft_ref, bf_ref,
                wt1_ref, wt2_ref, btc_ref, xgoldT_ref, out_ref,
                m_ref, d_ref, s_ref):
    i = pl.program_id(0)
    nsteps = pl.num_programs(0)
    NB = x_ref.shape[0]
    B = m_ref.shape[1]

    @pl.when(i == 0)
    def _init():
        m_ref[...] = jnp.full_like(m_ref, -jnp.inf)
        d_ref[...] = jnp.zeros_like(d_ref)
        s_ref[...] = jnp.zeros_like(s_ref)

    xb = x_ref[...]
    bcol = batch_ref[...].reshape(NB, 1)
    onehot = bcol == lax.broadcasted_iota(jnp.int32, (NB, B), 1)

    gate = jnp.sum(xb * wg_ref[...], axis=1, keepdims=True) + bg_ref[...]

    bm = jnp.max(jnp.where(onehot, gate, -jnp.inf), axis=0, keepdims=True)
    m_old = m_ref[...]
    m_new = jnp.maximum(m_old, bm)
    m_gat = jnp.sum(jnp.where(onehot, m_new, 0.0), axis=1, keepdims=True)
    eb = jnp.exp(gate - m_gat)
    d_contrib = jnp.sum(jnp.where(onehot, eb, 0.0), axis=0, keepdims=True)
    r = jnp.where(m_new == -jnp.inf, 0.0, jnp.exp(m_old - m_new))

    feat = jnp.dot(xb.astype(jnp.bfloat16), wft_ref[...],
                   preferred_element_type=jnp.float32) + bf_ref[...]
    feat = _leaky(feat)
    wfeat = (feat * eb).astype(jnp.bfloat16)
    oh_bf = onehot.astype(jnp.bfloat16)
    s_contrib = lax.dot_general(wfeat, oh_bf, (((0,), (0,)), ((), ())),
                                preferred_element_type=jnp.float32)

    m_ref[...] = m_new
    d_ref[...] = r * d_ref[...] + d_contrib
    s_ref[...] = r * s_ref[...] + s_contrib

    @pl.when(i == nsteps - 1)
    def _epilogue():
        invd = 1.0 / jnp.maximum(d_ref[...], 1e-16)
        xgT = s_ref[...] * invd
        xgoldT = xgoldT_ref[...]
        pre = (jnp.dot(wt1_ref[...], xgT, preferred_element_type=jnp.float32)
               + jnp.dot(wt2_ref[...], xgoldT, preferred_element_type=jnp.float32)
               + btc_ref[...])
        out_ref[...] = _leaky(pre) + xgoldT


def kernel(xg_old, x, batch, Wg, bg, Wf, bf, Wt, bt):
    N, D = x.shape
    B = xg_old.shape[0]
    NB = 2000
    G = N // NB

    batch3 = batch.astype(jnp.int32).reshape(G, NB, 1)
    WfT = Wf.T.astype(jnp.bfloat16)
    Wt1 = Wt[:, :D]
    Wt2 = Wt[:, D:]
    xg_oldT = xg_old.T
    bf_row = bf.reshape(1, D)
    bt_col = bt.reshape(D, 1)
    bg11 = bg.reshape(1, 1)

    out_T = pl.pallas_call(
        _fused_body,
        grid=(G,),
        in_specs=[
            pl.BlockSpec((NB, D), lambda i: (i, 0)),
            pl.BlockSpec((1, NB, 1), lambda i: (i, 0, 0)),
            pl.BlockSpec((1, D), lambda i: (0, 0)),
            pl.BlockSpec((1, 1), lambda i: (0, 0)),
            pl.BlockSpec((D, D), lambda i: (0, 0)),
            pl.BlockSpec((1, D), lambda i: (0, 0)),
            pl.BlockSpec((D, D), lambda i: (0, 0)),
            pl.BlockSpec((D, D), lambda i: (0, 0)),
            pl.BlockSpec((D, 1), lambda i: (0, 0)),
            pl.BlockSpec((D, B), lambda i: (0, 0)),
        ],
        out_specs=pl.BlockSpec((D, B), lambda i: (0, 0)),
        out_shape=jax.ShapeDtypeStruct((D, B), jnp.float32),
        scratch_shapes=[
            pltpu.VMEM((1, B), jnp.float32),
            pltpu.VMEM((1, B), jnp.float32),
            pltpu.VMEM((D, B), jnp.float32),
        ],
    )(x, batch3, Wg, bg11, WfT, bf_row, Wt1, Wt2, bt_col, xg_oldT)
    return out_T.T


# speedup vs baseline: 10.8436x; 1.0272x over previous
import jax
import jax.numpy as jnp
from jax import lax
from jax.experimental import pallas as pl
from jax.experimental.pallas import tpu as pltpu


def _leaky(v):
    return jnp.where(v >= 0, v, 0.01 * v)


_NEG = -(2.0 ** 100)


def _fused_body(x_ref, batch_ref, wg_ref, bg_ref, wft_ref, bf_ref,
                wt1_ref, wt2_ref, btc_ref, xgoldT_ref, out_ref,
                m_ref, d_ref, s_ref):
    i = pl.program_id(0)
    nsteps = pl.num_programs(0)
    NB = x_ref.shape[0]
    B = m_ref.shape[1]

    @pl.when(i == 0)
    def _init():
        m_ref[...] = jnp.full_like(m_ref, _NEG)
        d_ref[...] = jnp.zeros_like(d_ref)
        s_ref[...] = jnp.zeros_like(s_ref)

    xb_bf = x_ref[...].astype(jnp.bfloat16)
    bcol = batch_ref[...].reshape(NB, 1)
    onehot = bcol == lax.broadcasted_iota(jnp.int32, (NB, B), 1)

    gate = jnp.dot(xb_bf, wg_ref[...],
                   preferred_element_type=jnp.float32) + bg_ref[...]

    bm = jnp.max(jnp.where(onehot, gate, _NEG), axis=0, keepdims=True)
    m_old = m_ref[...]
    m_newf = jnp.maximum(m_old, bm)
    m_gat = jnp.sum(jnp.where(onehot, m_newf, 0.0), axis=1, keepdims=True)
    eb = jnp.exp(gate - m_gat)
    r = jnp.exp(m_old - m_newf)

    ohw = jnp.where(onehot, eb, 0.0).astype(jnp.bfloat16)
    ones = jnp.ones((NB, 1), dtype=jnp.bfloat16)
    d_contrib = lax.dot_general(ones, ohw, (((0,), (0,)), ((), ())),
                                preferred_element_type=jnp.float32)

    feat = jnp.dot(xb_bf, wft_ref[...],
                   preferred_element_type=jnp.float32) + bf_ref[...]
    feat_bf = _leaky(feat).astype(jnp.bfloat16)
    s_contrib = lax.dot_general(feat_bf, ohw, (((0,), (0,)), ((), ())),
                                preferred_element_type=jnp.float32)

    m_ref[...] = m_newf
    d_ref[...] = r * d_ref[...] + d_contrib
    s_ref[...] = r * s_ref[...] + s_contrib

    @pl.when(i == nsteps - 1)
    def _epilogue():
        invd = 1.0 / jnp.maximum(d_ref[...], 1e-16)
        xgT = s_ref[...] * invd
        xgoldT = xgoldT_ref[...]
        pre = (jnp.dot(wt1_ref[...], xgT, preferred_element_type=jnp.float32)
               + jnp.dot(wt2_ref[...], xgoldT, preferred_element_type=jnp.float32)
               + btc_ref[...])
        out_ref[...] = _leaky(pre) + xgoldT


def kernel(xg_old, x, batch, Wg, bg, Wf, bf, Wt, bt):
    N, D = x.shape
    B = xg_old.shape[0]
    NB = 2000
    G = N // NB

    batch3 = batch.astype(jnp.int32).reshape(G, NB, 1)
    WgT_bf = Wg.T.astype(jnp.bfloat16)
    WfT = Wf.T.astype(jnp.bfloat16)
    Wt1 = Wt[:, :D]
    Wt2 = Wt[:, D:]
    xg_oldT = xg_old.T
    bf_row = bf.reshape(1, D)
    bt_col = bt.reshape(D, 1)
    bg11 = bg.reshape(1, 1)

    out_T = pl.pallas_call(
        _fused_body,
        grid=(G,),
        in_specs=[
            pl.BlockSpec((NB, D), lambda i: (i, 0)),
            pl.BlockSpec((1, NB, 1), lambda i: (i, 0, 0)),
            pl.BlockSpec((D, 1), lambda i: (0, 0)),
            pl.BlockSpec((1, 1), lambda i: (0, 0)),
            pl.BlockSpec((D, D), lambda i: (0, 0)),
            pl.BlockSpec((1, D), lambda i: (0, 0)),
            pl.BlockSpec((D, D), lambda i: (0, 0)),
            pl.BlockSpec((D, D), lambda i: (0, 0)),
            pl.BlockSpec((D, 1), lambda i: (0, 0)),
            pl.BlockSpec((D, B), lambda i: (0, 0)),
        ],
        out_specs=pl.BlockSpec((D, B), lambda i: (0, 0)),
        out_shape=jax.ShapeDtypeStruct((D, B), jnp.float32),
        scratch_shapes=[
            pltpu.VMEM((1, B), jnp.float32),
            pltpu.VMEM((1, B), jnp.float32),
            pltpu.VMEM((D, B), jnp.float32),
        ],
    )(x, batch3, WgT_bf, bg11, WfT, bf_row, Wt1, Wt2, bt_col, xg_oldT)
    return out_T.T


# speedup vs baseline: 12.2738x; 1.1319x over previous
import jax
import jax.numpy as jnp
from jax import lax
from jax.experimental import pallas as pl
from jax.experimental.pallas import tpu as pltpu


def _leaky(v):
    return jnp.where(v >= 0, v, 0.01 * v)


def _fused_body(x_ref, batch_ref, wg_ref, bg_ref, wft_ref, bf_ref,
                wt1_ref, wt2_ref, btc_ref, xgoldT_ref, out_ref,
                d_ref, s_ref):
    i = pl.program_id(0)
    nsteps = pl.num_programs(0)
    NB = x_ref.shape[0]
    B = d_ref.shape[1]

    @pl.when(i == 0)
    def _init():
        d_ref[...] = jnp.zeros_like(d_ref)
        s_ref[...] = jnp.zeros_like(s_ref)

    xb_bf = x_ref[...].astype(jnp.bfloat16)
    bcol = batch_ref[...].reshape(NB, 1)
    onehot = bcol == lax.broadcasted_iota(jnp.int32, (NB, B), 1)

    gate = jnp.dot(xb_bf, wg_ref[...],
                   preferred_element_type=jnp.float32) + bg_ref[...]
    eb = jnp.exp(jnp.clip(gate, -60.0, 60.0))

    ohw = jnp.where(onehot, eb, 0.0).astype(jnp.bfloat16)
    ones = jnp.ones((NB, 1), dtype=jnp.bfloat16)
    d_contrib = lax.dot_general(ones, ohw, (((0,), (0,)), ((), ())),
                                preferred_element_type=jnp.float32)

    feat = jnp.dot(xb_bf, wft_ref[...],
                   preferred_element_type=jnp.float32).astype(jnp.bfloat16)
    feat_bf = _leaky(feat + bf_ref[...])
    s_contrib = lax.dot_general(feat_bf, ohw, (((0,), (0,)), ((), ())),
                                preferred_element_type=jnp.float32)

    d_ref[...] += d_contrib
    s_ref[...] += s_contrib

    @pl.when(i == nsteps - 1)
    def _epilogue():
        invd = 1.0 / jnp.maximum(d_ref[...], 1e-16)
        xgT = s_ref[...] * invd
        xgoldT = xgoldT_ref[...]
        pre = (jnp.dot(wt1_ref[...], xgT, preferred_element_type=jnp.float32)
               + jnp.dot(wt2_ref[...], xgoldT, preferred_element_type=jnp.float32)
               + btc_ref[...])
        out_ref[...] = _leaky(pre) + xgoldT


def kernel(xg_old, x, batch, Wg, bg, Wf, bf, Wt, bt):
    N, D = x.shape
    B = xg_old.shape[0]
    NB = 2000
    G = N // NB

    batch3 = batch.astype(jnp.int32).reshape(G, NB, 1)
    WgT_bf = Wg.T.astype(jnp.bfloat16)
    WfT = Wf.T.astype(jnp.bfloat16)
    Wt1 = Wt[:, :D]
    Wt2 = Wt[:, D:]
    xg_oldT = xg_old.T
    bf_row = bf.reshape(1, D).astype(jnp.bfloat16)
    bt_col = bt.reshape(D, 1)
    bg11 = bg.reshape(1, 1)

    out_T = pl.pallas_call(
        _fused_body,
        grid=(G,),
        in_specs=[
            pl.BlockSpec((NB, D), lambda i: (i, 0)),
            pl.BlockSpec((1, NB, 1), lambda i: (i, 0, 0)),
            pl.BlockSpec((D, 1), lambda i: (0, 0)),
            pl.BlockSpec((1, 1), lambda i: (0, 0)),
            pl.BlockSpec((D, D), lambda i: (0, 0)),
            pl.BlockSpec((1, D), lambda i: (0, 0)),
            pl.BlockSpec((D, D), lambda i: (0, 0)),
            pl.BlockSpec((D, D), lambda i: (0, 0)),
            pl.BlockSpec((D, 1), lambda i: (0, 0)),
            pl.BlockSpec((D, B), lambda i: (0, 0)),
        ],
        out_specs=pl.BlockSpec((D, B), lambda i: (0, 0)),
        out_shape=jax.ShapeDtypeStruct((D, B), jnp.float32),
        scratch_shapes=[
            pltpu.VMEM((1, B), jnp.float32),
            pltpu.VMEM((D, B), jnp.float32),
        ],
    )(x, batch3, WgT_bf, bg11, WfT, bf_row, Wt1, Wt2, bt_col, xg_oldT)
    return out_T.T


# speedup vs baseline: 22.0848x; 1.7993x over previous
import jax
import jax.numpy as jnp
from jax import lax
from jax.experimental import pallas as pl
from jax.experimental.pallas import tpu as pltpu


def _leaky(v):
    return jnp.where(v >= 0, v, 0.01 * v)


def _fused_body(x_ref, batch_ref, wg_ref, bg_ref, wft_ref, bf_ref,
                wt1_ref, wt2_ref, bt_ref, xgold_ref, out_ref,
                d_ref, s_ref):
    i = pl.program_id(0)
    nsteps = pl.num_programs(0)
    NB = x_ref.shape[0]
    B = d_ref.shape[0]

    @pl.when(i == 0)
    def _init():
        d_ref[...] = jnp.zeros_like(d_ref)
        s_ref[...] = jnp.zeros_like(s_ref)

    xb_bf = x_ref[...].astype(jnp.bfloat16)
    brow = batch_ref[...].reshape(1, NB)

    gate = lax.dot_general(wg_ref[...], xb_bf, (((0,), (1,)), ((), ())),
                           preferred_element_type=jnp.float32) + bg_ref[...]
    eb = jnp.exp(jnp.clip(gate, -60.0, 60.0))

    onehot = brow == lax.broadcasted_iota(jnp.int32, (B, NB), 0)
    ohw = jnp.where(onehot, eb, 0.0).astype(jnp.bfloat16)
    ones = jnp.ones((NB, 1), dtype=jnp.bfloat16)
    d_contrib = jnp.dot(ohw, ones,
                        preferred_element_type=jnp.float32)

    feat = jnp.dot(xb_bf, wft_ref[...],
                   preferred_element_type=jnp.float32).astype(jnp.bfloat16)
    feat_bf = _leaky(feat + bf_ref[...])
    s_contrib = jnp.dot(ohw, feat_bf,
                        preferred_element_type=jnp.float32)

    d_ref[...] += d_contrib
    s_ref[...] += s_contrib

    @pl.when(i == nsteps - 1)
    def _epilogue():
        invd = 1.0 / jnp.maximum(d_ref[...], 1e-16)
        xg = s_ref[...] * invd
        xgold = xgold_ref[...]
        pre = (jnp.dot(xg, wt1_ref[...], preferred_element_type=jnp.float32)
               + jnp.dot(xgold, wt2_ref[...], preferred_element_type=jnp.float32)
               + bt_ref[...])
        out_ref[...] = _leaky(pre) + xgold


def kernel(xg_old, x, batch, Wg, bg, Wf, bf, Wt, bt):
    N, D = x.shape
    B = xg_old.shape[0]
    NB = 2000
    G = N // NB

    batch3 = batch.astype(jnp.int32).reshape(G, 1, NB)
    WgT_bf = Wg.T.astype(jnp.bfloat16)
    WfT = Wf.T.astype(jnp.bfloat16)
    Wt1T = Wt[:, :D].T
    Wt2T = Wt[:, D:].T
    bf_row = bf.reshape(1, D).astype(jnp.bfloat16)
    bt_row = bt.reshape(1, D)
    bg11 = bg.reshape(1, 1)

    out = pl.pallas_call(
        _fused_body,
        grid=(G,),
        in_specs=[
            pl.BlockSpec((NB, D), lambda i: (i, 0)),
            pl.BlockSpec((1, 1, NB), lambda i: (i, 0, 0)),
            pl.BlockSpec((D, 1), lambda i: (0, 0)),
            pl.BlockSpec((1, 1), lambda i: (0, 0)),
            pl.BlockSpec((D, D), lambda i: (0, 0)),
            pl.BlockSpec((1, D), lambda i: (0, 0)),
            pl.BlockSpec((D, D), lambda i: (0, 0)),
            pl.BlockSpec((D, D), lambda i: (0, 0)),
            pl.BlockSpec((1, D), lambda i: (0, 0)),
            pl.BlockSpec((B, D), lambda i: (0, 0)),
        ],
        out_specs=pl.BlockSpec((B, D), lambda i: (0, 0)),
        out_shape=jax.ShapeDtypeStruct((B, D), jnp.float32),
        scratch_shapes=[
            pltpu.VMEM((B, 1), jnp.float32),
            pltpu.VMEM((B, D), jnp.float32),
        ],
    )(x, batch3, WgT_bf, bg11, WfT, bf_row, Wt1T, Wt2T, bt_row, xg_old)
    return out


# speedup vs baseline: 27.9005x; 1.2633x over previous
import jax
import jax.numpy as jnp
from jax import lax
from jax.experimental import pallas as pl
from jax.experimental.pallas import tpu as pltpu


def _leaky(v):
    return jnp.where(v >= 0, v, 0.01 * v)


def _fused_body(x_ref, batch_ref, wg_ref, bg_ref, wft_ref, bf_ref,
                wt1_ref, wt2_ref, bt_ref, xgold_ref, out_ref,
                d_ref, s_ref):
    i = pl.program_id(0)
    nsteps = pl.num_programs(0)
    NB = x_ref.shape[0]
    B = d_ref.shape[0]

    @pl.when(i == 0)
    def _init():
        d_ref[...] = jnp.zeros_like(d_ref)
        s_ref[...] = jnp.zeros_like(s_ref)

    xb_bf = x_ref[...].astype(jnp.bfloat16)
    brow = batch_ref[...].reshape(1, NB)

    gate = lax.dot_general(wg_ref[...], xb_bf, (((0,), (1,)), ((), ())),
                           preferred_element_type=jnp.float32) + bg_ref[...]
    eb = jnp.exp(jnp.clip(gate, -60.0, 60.0))

    onehot = brow == lax.broadcasted_iota(jnp.int32, (B, NB), 0)
    ohw = jnp.where(onehot, eb, 0.0).astype(jnp.bfloat16)
    ones = jnp.ones((NB, 1), dtype=jnp.bfloat16)
    d_contrib = jnp.dot(ohw, ones,
                        preferred_element_type=jnp.float32)

    feat = jnp.dot(xb_bf, wft_ref[...],
                   preferred_element_type=jnp.float32).astype(jnp.bfloat16)
    feat_bf = _leaky(feat + bf_ref[...])
    s_contrib = jnp.dot(ohw, feat_bf,
                        preferred_element_type=jnp.float32)

    d_ref[...] += d_contrib
    s_ref[...] += s_contrib

    @pl.when(i == nsteps - 1)
    def _epilogue():
        invd = 1.0 / jnp.maximum(d_ref[...], 1e-16)
        xg = s_ref[...] * invd
        xgold = xgold_ref[...]
        pre = (jnp.dot(xg, wt1_ref[...], preferred_element_type=jnp.float32)
               + jnp.dot(xgold, wt2_ref[...], preferred_element_type=jnp.float32)
               + bt_ref[...])
        out_ref[...] = _leaky(pre) + xgold


def kernel(xg_old, x, batch, Wg, bg, Wf, bf, Wt, bt):
    N, D = x.shape
    B = xg_old.shape[0]
    NB = 5000
    G = N // NB

    batch3 = batch.astype(jnp.int32).reshape(G, 1, NB)
    WgT_bf = Wg.T.astype(jnp.bfloat16)
    WfT = Wf.T.astype(jnp.bfloat16)
    Wt1T = Wt[:, :D].T
    Wt2T = Wt[:, D:].T
    bf_row = bf.reshape(1, D).astype(jnp.bfloat16)
    bt_row = bt.reshape(1, D)
    bg11 = bg.reshape(1, 1)

    out = pl.pallas_call(
        _fused_body,
        grid=(G,),
        in_specs=[
            pl.BlockSpec((NB, D), lambda i: (i, 0)),
            pl.BlockSpec((1, 1, NB), lambda i: (i, 0, 0)),
            pl.BlockSpec((D, 1), lambda i: (0, 0)),
            pl.BlockSpec((1, 1), lambda i: (0, 0)),
            pl.BlockSpec((D, D), lambda i: (0, 0)),
            pl.BlockSpec((1, D), lambda i: (0, 0)),
            pl.BlockSpec((D, D), lambda i: (0, 0)),
            pl.BlockSpec((D, D), lambda i: (0, 0)),
            pl.BlockSpec((1, D), lambda i: (0, 0)),
            pl.BlockSpec((B, D), lambda i: (0, 0)),
        ],
        out_specs=pl.BlockSpec((B, D), lambda i: (0, 0)),
        out_shape=jax.ShapeDtypeStruct((B, D), jnp.float32),
        scratch_shapes=[
            pltpu.VMEM((B, 1), jnp.float32),
            pltpu.VMEM((B, D), jnp.float32),
        ],
    )(x, batch3, WgT_bf, bg11, WfT, bf_row, Wt1T, Wt2T, bt_row, xg_old)
    return out
